# baseline (speedup 1.0000x reference)
import functools

import jax
import jax.numpy as jnp
from jax import lax
from jax.experimental import pallas as pl
from jax.experimental.pallas import tpu as pltpu
from jax.experimental.pallas import tpu_sc as plsc

N = 10000
E = 320000
D = 128
DA = 144
NC, NS = 2, 16
NW = NC * NS
EPT = E // NW
CH = 80
NBUF = 3
G = 25
ECH = E // CH
NSUP0, NSUP1 = 6, 4
CPT0 = NSUP0 * G
CPT1 = NSUP1 * G
ROWS_PT = N // NS


def _sc_segment_sum(xa, srcr, dstr, zrows):
    mesh = plsc.VectorSubcoreMesh(core_axis_name="c", subcore_axis_name="s")

    @functools.partial(
        pl.kernel,
        mesh=mesh,
        out_type=jax.ShapeDtypeStruct((NC, NS, ROWS_PT, DA), jnp.float32),
        scratch_types=[
            pltpu.VMEM((G, CH), jnp.int32),
            pltpu.VMEM((G, CH), jnp.int32),
            pltpu.VMEM((NBUF, CH, DA), jnp.float32),
            pltpu.VMEM_SHARED((N, DA), jnp.float32),
        ] + [pltpu.SemaphoreType.DMA] * NBUF,
        compiler_params=pltpu.CompilerParams(use_tc_tiling_on_sc=False),
    )
    def k(xa_hbm, src_hbm, dst_hbm, z_hbm, part_hbm, src_v, dst_v, rows,
          acc_sh, *sems):
        c = lax.axis_index("c")
        s = lax.axis_index("s")
        cbase = jnp.where(c == 0, s * CPT0, NS * CPT0 + s * CPT1)
        nsup = jnp.where(c == 0, NSUP0, NSUP1)

        pltpu.sync_copy(z_hbm, acc_sh.at[pl.ds(s * ROWS_PT, ROWS_PT)])
        plsc.subcore_barrier()

        def superchunk(g5, carry):
            pltpu.sync_copy(src_hbm.at[pl.ds(cbase + g5 * G, G)], src_v)
            pltpu.sync_copy(dst_hbm.at[pl.ds(cbase + g5 * G, G)], dst_v)
            for b in range(NBUF):
                pltpu.async_copy(xa_hbm.at[src_v.at[b]], rows.at[b], sems[b])

            def chunk(i, carry2):
                for b in range(NBUF):

                    @pl.when(i % NBUF == b)
                    def _():
                        pltpu.make_async_copy(xa_hbm.at[src_v.at[i]],
                                              rows.at[b], sems[b]).wait()
                        pltpu.sync_copy(rows.at[b], acc_sh.at[dst_v.at[i]],
                                        add=True)

                        @pl.when(i + NBUF < G)
                        def _():
                            pltpu.async_copy(xa_hbm.at[src_v.at[i + NBUF]],
                                             rows.at[b], sems[b])
                return carry2

            lax.fori_loop(0, G, chunk, 0)
            return carry

        lax.fori_loop(0, nsup, superchunk, 0)

        plsc.subcore_barrier()
        pltpu.sync_copy(acc_sh.at[pl.ds(s * ROWS_PT, ROWS_PT)],
                        part_hbm.at[c, s])

    return k(xa, srcr, dstr, zrows)


def _tc_finish(parts, x, wlt, wrt, b):
    B = 1000

    def body(p_ref, x_ref, wlt_ref, wrt_ref, b_ref, o_ref):
        p = p_ref[...]
        ssum = p[0] + p[1]
        summed = ssum[:, :D]
        cnt = jnp.sum(ssum[:, D:], axis=1, keepdims=True)
        mean = summed / jnp.maximum(cnt, 1.0)
        o_ref[...] = (
            jnp.dot(mean, wlt_ref[...], preferred_element_type=jnp.float32)
            + jnp.dot(x_ref[...], wrt_ref[...],
                      preferred_element_type=jnp.float32)
            + b_ref[...]
        )

    return pl.pallas_call(
        body,
        grid=(N // B,),
        in_specs=[
            pl.BlockSpec((NC, B, DA), lambda i: (0, i, 0)),
            pl.BlockSpec((B, D), lambda i: (i, 0)),
            pl.BlockSpec((D, D), lambda i: (0, 0)),
            pl.BlockSpec((D, D), lambda i: (0, 0)),
            pl.BlockSpec((1, D), lambda i: (0, 0)),
        ],
        out_specs=pl.BlockSpec((B, D), lambda i: (i, 0)),
        out_shape=jax.ShapeDtypeStruct((N, D), jnp.float32),
    )(parts, x, wlt, wrt, b)


def kernel(x, edge_index, W_l, b_l, W_r, training):
    xa = jnp.concatenate(
        [x, jnp.ones((N, 1), jnp.float32), jnp.zeros((N, DA - D - 1),
                                                     jnp.float32)], axis=1)
    src = edge_index[0].astype(jnp.int32).reshape(ECH, CH)
    dst = edge_index[1].astype(jnp.int32).reshape(ECH, CH)
    zrows = jnp.zeros((ROWS_PT, DA), jnp.float32)
    parts = _sc_segment_sum(xa, src, dst, zrows)
    parts = parts.reshape(NC, N, DA)
    return _tc_finish(parts, x, W_l.T, W_r.T, b_l.reshape(1, D))


# speedup vs baseline: 1.0031x; 1.0031x over previous
import functools

import jax
import jax.numpy as jnp
from jax import lax
from jax.experimental import pallas as pl
from jax.experimental.pallas import tpu as pltpu
from jax.experimental.pallas import tpu_sc as plsc

N = 10000
E = 320000
D = 128
DA = 144
NC, NS = 2, 16
NW = NC * NS
EPT = E // NW
CH = 80
NBUF = 3
G = 25
ECH = E // CH
NSUP0, NSUP1 = 6, 4
CPT0 = NSUP0 * G
CPT1 = NSUP1 * G
ROWS_PT = N // NS


def _sc_segment_sum(xa, srcr, dstr, zrows):
    mesh = plsc.VectorSubcoreMesh(core_axis_name="c", subcore_axis_name="s")

    @functools.partial(
        pl.kernel,
        mesh=mesh,
        out_type=jax.ShapeDtypeStruct((NC, NS, ROWS_PT, DA), jnp.float32),
        scratch_types=[
            pltpu.VMEM((G, CH), jnp.int32),
            pltpu.VMEM((G, CH), jnp.int32),
            pltpu.VMEM((NBUF, CH, DA), jnp.float32),
            pltpu.VMEM_SHARED((N, DA), jnp.float32),
        ] + [pltpu.SemaphoreType.DMA] * NBUF,
        compiler_params=pltpu.CompilerParams(use_tc_tiling_on_sc=False),
    )
    def k(xa_hbm, src_hbm, dst_hbm, z_hbm, part_hbm, src_v, dst_v, rows,
          acc_sh, *sems):
        c = lax.axis_index("c")
        s = lax.axis_index("s")
        cbase = jnp.where(c == 1, s * CPT0, NS * CPT0 + s * CPT1)
        nsup = jnp.where(c == 1, NSUP0, NSUP1)

        pltpu.sync_copy(z_hbm, acc_sh.at[pl.ds(s * ROWS_PT, ROWS_PT)])
        plsc.subcore_barrier()

        def superchunk(g5, carry):
            pltpu.sync_copy(src_hbm.at[pl.ds(cbase + g5 * G, G)], src_v)
            pltpu.sync_copy(dst_hbm.at[pl.ds(cbase + g5 * G, G)], dst_v)
            for b in range(NBUF):
                pltpu.async_copy(xa_hbm.at[src_v.at[b]], rows.at[b], sems[b])

            def chunk(i, carry2):
                for b in range(NBUF):

                    @pl.when(i % NBUF == b)
                    def _():
                        pltpu.make_async_copy(xa_hbm.at[src_v.at[i]],
                                              rows.at[b], sems[b]).wait()
                        pltpu.sync_copy(rows.at[b], acc_sh.at[dst_v.at[i]],
                                        add=True)

                        @pl.when(i + NBUF < G)
                        def _():
                            pltpu.async_copy(xa_hbm.at[src_v.at[i + NBUF]],
                                             rows.at[b], sems[b])
                return carry2

            lax.fori_loop(0, G, chunk, 0)
            return carry

        lax.fori_loop(0, nsup, superchunk, 0)

        plsc.subcore_barrier()
        pltpu.sync_copy(acc_sh.at[pl.ds(s * ROWS_PT, ROWS_PT)],
                        part_hbm.at[c, s])

    return k(xa, srcr, dstr, zrows)


def _tc_finish(parts, x, wlt, wrt, b):
    B = 1000

    def body(p_ref, x_ref, wlt_ref, wrt_ref, b_ref, o_ref):
        p = p_ref[...]
        ssum = p[0] + p[1]
        summed = ssum[:, :D]
        cnt = jnp.sum(ssum[:, D:], axis=1, keepdims=True)
        mean = summed / jnp.maximum(cnt, 1.0)
        o_ref[...] = (
            jnp.dot(mean, wlt_ref[...], preferred_element_type=jnp.float32)
            + jnp.dot(x_ref[...], wrt_ref[...],
                      preferred_element_type=jnp.float32)
            + b_ref[...]
        )

    return pl.pallas_call(
        body,
        grid=(N // B,),
        in_specs=[
            pl.BlockSpec((NC, B, DA), lambda i: (0, i, 0)),
            pl.BlockSpec((B, D), lambda i: (i, 0)),
            pl.BlockSpec((D, D), lambda i: (0, 0)),
            pl.BlockSpec((D, D), lambda i: (0, 0)),
            pl.BlockSpec((1, D), lambda i: (0, 0)),
        ],
        out_specs=pl.BlockSpec((B, D), lambda i: (i, 0)),
        out_shape=jax.ShapeDtypeStruct((N, D), jnp.float32),
    )(parts, x, wlt, wrt, b)


def kernel(x, edge_index, W_l, b_l, W_r, training):
    xa = jnp.concatenate(
        [x, jnp.ones((N, 1), jnp.float32), jnp.zeros((N, DA - D - 1),
                                                     jnp.float32)], axis=1)
    src = edge_index[0].astype(jnp.int32).reshape(ECH, CH)
    dst = edge_index[1].astype(jnp.int32).reshape(ECH, CH)
    zrows = jnp.zeros((ROWS_PT, DA), jnp.float32)
    parts = _sc_segment_sum(xa, src, dst, zrows)
    parts = parts.reshape(NC, N, DA)
    return _tc_finish(parts, x, W_l.T, W_r.T, b_l.reshape(1, D))


# speedup vs baseline: 1.3539x; 1.3496x over previous
import functools

import jax
import jax.numpy as jnp
from jax import lax
from jax.experimental import pallas as pl
from jax.experimental.pallas import tpu as pltpu
from jax.experimental.pallas import tpu_sc as plsc

N = 10000
E = 320000
D = 128
NC, NS = 2, 16
NW = NC * NS
EPT = E // NW
CH = 80
NCHUNK = EPT // CH
NBUF = 3
G = 25
NSUP = NCHUNK // G
ROWS_PT = N // NS


def _sc_segment_sum(x, srcr, dstr, zrows, zcnt):
    mesh = plsc.VectorSubcoreMesh(core_axis_name="c", subcore_axis_name="s")

    @functools.partial(
        pl.kernel,
        mesh=mesh,
        out_type=(
            jax.ShapeDtypeStruct((NC, NS, ROWS_PT, D), jnp.float32),
            jax.ShapeDtypeStruct((NC, NS, N), jnp.float32),
        ),
        scratch_types=[
            pltpu.VMEM((G, CH), jnp.int32),
            pltpu.VMEM((G, CH), jnp.int32),
            pltpu.VMEM((NBUF, CH, D), jnp.float32),
            pltpu.VMEM((N,), jnp.float32),
            pltpu.VMEM_SHARED((N, D), jnp.float32),
        ] + [pltpu.SemaphoreType.DMA] * NBUF,
        compiler_params=pltpu.CompilerParams(use_tc_tiling_on_sc=False,
                                             needs_layout_passes=False),
    )
    def k(x_hbm, src_hbm, dst_hbm, z_hbm, zc_hbm, part_hbm, cnt_hbm,
          src_v, dst_v, rows, hist, acc_sh, *sems):
        c = lax.axis_index("c")
        s = lax.axis_index("s")
        wid = c * NS + s
        ones16 = jnp.ones((16,), jnp.float32)

        pltpu.sync_copy(z_hbm, acc_sh.at[pl.ds(s * ROWS_PT, ROWS_PT)])
        pltpu.sync_copy(zc_hbm, hist)
        plsc.subcore_barrier()

        def superchunk(g5, carry):
            pltpu.sync_copy(src_hbm.at[wid, pl.ds(g5 * G, G)], src_v)
            pltpu.sync_copy(dst_hbm.at[wid, pl.ds(g5 * G, G)], dst_v)
            for b in range(NBUF):
                pltpu.async_copy(x_hbm.at[src_v.at[b]], rows.at[b], sems[b])

            def chunk(i, carry2):
                for b in range(NBUF):

                    @pl.when(i % NBUF == b)
                    def _():
                        pltpu.make_async_copy(x_hbm.at[src_v.at[i]],
                                              rows.at[b], sems[b]).wait()
                        pltpu.sync_copy(rows.at[b], acc_sh.at[dst_v.at[i]],
                                        add=True)

                        @pl.when(i + NBUF < G)
                        def _():
                            pltpu.async_copy(x_hbm.at[src_v.at[i + NBUF]],
                                             rows.at[b], sems[b])
                for kk in range(CH // 16):
                    idx = dst_v[i, pl.ds(kk * 16, 16)]
                    plsc.addupdate_scatter(hist, [idx], ones16)
                return carry2

            lax.fori_loop(0, G, chunk, 0)
            return carry

        lax.fori_loop(0, NSUP, superchunk, 0)

        pltpu.sync_copy(hist, cnt_hbm.at[c, s])
        plsc.subcore_barrier()
        pltpu.sync_copy(acc_sh.at[pl.ds(s * ROWS_PT, ROWS_PT)],
                        part_hbm.at[c, s])

    return k(x, srcr, dstr, zrows, zcnt)


def _tc_finish(parts, cnts, x, wlt, wrt, b):
    B = 1000

    def body(p_ref, c_ref, x_ref, wlt_ref, wrt_ref, b_ref, o_ref):
        p = p_ref[...]
        summed = p[0] + p[1]
        cnt = jnp.sum(c_ref[...], axis=1, keepdims=True)
        mean = summed / jnp.maximum(cnt, 1.0)
        o_ref[...] = (
            jnp.dot(mean, wlt_ref[...], preferred_element_type=jnp.float32)
            + jnp.dot(x_ref[...], wrt_ref[...],
                      preferred_element_type=jnp.float32)
            + b_ref[...]
        )

    return pl.pallas_call(
        body,
        grid=(N // B,),
        in_specs=[
            pl.BlockSpec((NC, B, D), lambda i: (0, i, 0)),
            pl.BlockSpec((B, NW), lambda i: (i, 0)),
            pl.BlockSpec((B, D), lambda i: (i, 0)),
            pl.BlockSpec((D, D), lambda i: (0, 0)),
            pl.BlockSpec((D, D), lambda i: (0, 0)),
            pl.BlockSpec((1, D), lambda i: (0, 0)),
        ],
        out_specs=pl.BlockSpec((B, D), lambda i: (i, 0)),
        out_shape=jax.ShapeDtypeStruct((N, D), jnp.float32),
    )(parts, cnts, x, wlt, wrt, b)


def kernel(x, edge_index, W_l, b_l, W_r, training):
    src = edge_index[0].astype(jnp.int32).reshape(NW, NCHUNK, CH)
    dst = edge_index[1].astype(jnp.int32).reshape(NW, NCHUNK, CH)
    zrows = jnp.zeros((ROWS_PT, D), jnp.float32)
    zcnt = jnp.zeros((N,), jnp.float32)
    parts, cnts = _sc_segment_sum(x, src, dst, zrows, zcnt)
    parts = parts.reshape(NC, N, D)
    cnts = cnts.reshape(NW, N).T
    return _tc_finish(parts, cnts, x, W_l.T, W_r.T, b_l.reshape(1, D))


# speedup vs baseline: 1.4340x; 1.0592x over previous
import functools

import jax
import jax.numpy as jnp
from jax import lax
from jax.experimental import pallas as pl
from jax.experimental.pallas import tpu as pltpu
from jax.experimental.pallas import tpu_sc as plsc

N = 10000
E = 320000
D = 128
NC, NS = 2, 16
NW = NC * NS
EPT = E // NW
CH = 80
NCHUNK = EPT // CH
NBUF = 3
G = 25
NSUP = NCHUNK // G
ROWS_PT = N // NS


def _sc_segment_sum(x, srcr, dstr, zrows, zcnt):
    mesh = plsc.VectorSubcoreMesh(core_axis_name="c", subcore_axis_name="s")

    @functools.partial(
        pl.kernel,
        mesh=mesh,
        out_type=(
            jax.ShapeDtypeStruct((NC, NS, ROWS_PT, D), jnp.bfloat16),
            jax.ShapeDtypeStruct((NC, NS, N), jnp.float32),
        ),
        scratch_types=[
            pltpu.VMEM((G, CH), jnp.int32),
            pltpu.VMEM((G, CH), jnp.int32),
            pltpu.VMEM((NBUF, CH, D), jnp.bfloat16),
            pltpu.VMEM((N,), jnp.float32),
            pltpu.VMEM_SHARED((N, D), jnp.bfloat16),
        ] + [pltpu.SemaphoreType.DMA] * NBUF,
        compiler_params=pltpu.CompilerParams(use_tc_tiling_on_sc=False,
                                             needs_layout_passes=False),
    )
    def k(x_hbm, src_hbm, dst_hbm, z_hbm, zc_hbm, part_hbm, cnt_hbm,
          src_v, dst_v, rows, hist, acc_sh, *sems):
        c = lax.axis_index("c")
        s = lax.axis_index("s")
        wid = c * NS + s
        ones16 = jnp.ones((16,), jnp.float32)

        pltpu.sync_copy(z_hbm, acc_sh.at[pl.ds(s * ROWS_PT, ROWS_PT)])
        pltpu.sync_copy(zc_hbm, hist)
        plsc.subcore_barrier()

        def superchunk(g5, carry):
            pltpu.sync_copy(src_hbm.at[wid, pl.ds(g5 * G, G)], src_v)
            pltpu.sync_copy(dst_hbm.at[wid, pl.ds(g5 * G, G)], dst_v)
            for b in range(NBUF):
                pltpu.async_copy(x_hbm.at[src_v.at[b]], rows.at[b], sems[b])

            def chunk(i, carry2):
                for b in range(NBUF):

                    @pl.when(i % NBUF == b)
                    def _():
                        pltpu.make_async_copy(x_hbm.at[src_v.at[i]],
                                              rows.at[b], sems[b]).wait()
                        pltpu.sync_copy(rows.at[b], acc_sh.at[dst_v.at[i]],
                                        add=True)

                        @pl.when(i + NBUF < G)
                        def _():
                            pltpu.async_copy(x_hbm.at[src_v.at[i + NBUF]],
                                             rows.at[b], sems[b])
                for kk in range(CH // 16):
                    idx = dst_v[i, pl.ds(kk * 16, 16)]
                    plsc.addupdate_scatter(hist, [idx], ones16)
                return carry2

            lax.fori_loop(0, G, chunk, 0)
            return carry

        lax.fori_loop(0, NSUP, superchunk, 0)

        pltpu.sync_copy(hist, cnt_hbm.at[c, s])
        plsc.subcore_barrier()
        pltpu.sync_copy(acc_sh.at[pl.ds(s * ROWS_PT, ROWS_PT)],
                        part_hbm.at[c, s])

    return k(x, srcr, dstr, zrows, zcnt)


def _tc_finish(parts, cnts, x, wlt, wrt, b):
    B = 1000

    def body(p_ref, c_ref, x_ref, wlt_ref, wrt_ref, b_ref, o_ref):
        p = p_ref[...].astype(jnp.float32)
        summed = p[0] + p[1]
        cnt = jnp.sum(c_ref[...], axis=1, keepdims=True)
        mean = summed / jnp.maximum(cnt, 1.0)
        o_ref[...] = (
            jnp.dot(mean, wlt_ref[...], preferred_element_type=jnp.float32)
            + jnp.dot(x_ref[...], wrt_ref[...],
                      preferred_element_type=jnp.float32)
            + b_ref[...]
        )

    return pl.pallas_call(
        body,
        grid=(N // B,),
        in_specs=[
            pl.BlockSpec((NC, B, D), lambda i: (0, i, 0)),
            pl.BlockSpec((B, NW), lambda i: (i, 0)),
            pl.BlockSpec((B, D), lambda i: (i, 0)),
            pl.BlockSpec((D, D), lambda i: (0, 0)),
            pl.BlockSpec((D, D), lambda i: (0, 0)),
            pl.BlockSpec((1, D), lambda i: (0, 0)),
        ],
        out_specs=pl.BlockSpec((B, D), lambda i: (i, 0)),
        out_shape=jax.ShapeDtypeStruct((N, D), jnp.float32),
    )(parts, cnts, x, wlt, wrt, b)


def kernel(x, edge_index, W_l, b_l, W_r, training):
    src = edge_index[0].astype(jnp.int32).reshape(NW, NCHUNK, CH)
    dst = edge_index[1].astype(jnp.int32).reshape(NW, NCHUNK, CH)
    zrows = jnp.zeros((ROWS_PT, D), jnp.bfloat16)
    zcnt = jnp.zeros((N,), jnp.float32)
    xb = x.astype(jnp.bfloat16)
    parts, cnts = _sc_segment_sum(xb, src, dst, zrows, zcnt)
    parts = parts.reshape(NC, N, D)
    cnts = cnts.reshape(NW, N).T
    return _tc_finish(parts, cnts, x, W_l.T, W_r.T, b_l.reshape(1, D))


# speedup vs baseline: 1.6298x; 1.1366x over previous
import functools

import jax
import jax.numpy as jnp
from jax import lax
from jax.experimental import pallas as pl
from jax.experimental.pallas import tpu as pltpu
from jax.experimental.pallas import tpu_sc as plsc

N = 10000
E = 320000
D = 128
NC, NS = 2, 16
NW = NC * NS
EPT = E // NW
CH = 80
NCHUNK = EPT // CH
NBUF = 5
ROWS_PT = N // NS


def _sc_segment_sum(x, srcr, dstr, zrows, zcnt):
    mesh = plsc.VectorSubcoreMesh(core_axis_name="c", subcore_axis_name="s")

    @functools.partial(
        pl.kernel,
        mesh=mesh,
        out_type=(
            jax.ShapeDtypeStruct((NC, NS, ROWS_PT, D), jnp.bfloat16),
            jax.ShapeDtypeStruct((NC, NS, N), jnp.float32),
        ),
        scratch_types=[
            pltpu.VMEM((NCHUNK, CH), jnp.int32),
            pltpu.VMEM((NCHUNK, CH), jnp.int32),
            pltpu.VMEM((NBUF, CH, D), jnp.bfloat16),
            pltpu.VMEM((N,), jnp.float32),
            pltpu.VMEM_SHARED((N, D), jnp.bfloat16),
        ] + [pltpu.SemaphoreType.DMA] * (NBUF + 1),
        compiler_params=pltpu.CompilerParams(use_tc_tiling_on_sc=False,
                                             needs_layout_passes=False),
    )
    def k(x_hbm, src_hbm, dst_hbm, z_hbm, zc_hbm, part_hbm, cnt_hbm,
          src_v, dst_v, rows, hist, acc_sh, *sems):
        zsem = sems[NBUF]
        c = lax.axis_index("c")
        s = lax.axis_index("s")
        wid = c * NS + s
        ones16 = jnp.ones((16,), jnp.float32)

        pltpu.async_copy(z_hbm, acc_sh.at[pl.ds(s * ROWS_PT, ROWS_PT)], zsem)
        pltpu.sync_copy(src_hbm.at[wid], src_v)
        pltpu.sync_copy(dst_hbm.at[wid], dst_v)
        pltpu.sync_copy(zc_hbm, hist)
        for b in range(NBUF):
            pltpu.async_copy(x_hbm.at[src_v.at[b]], rows.at[b], sems[b])
        pltpu.make_async_copy(z_hbm, acc_sh.at[pl.ds(s * ROWS_PT, ROWS_PT)],
                              zsem).wait()
        plsc.subcore_barrier()

        def group(g, carry):
            for b in range(NBUF):
                j = g * NBUF + b
                pltpu.make_async_copy(x_hbm.at[src_v.at[j]], rows.at[b],
                                      sems[b]).wait()
                pltpu.sync_copy(rows.at[b], acc_sh.at[dst_v.at[j]], add=True)

                @pl.when(j + NBUF < NCHUNK)
                def _():
                    pltpu.async_copy(x_hbm.at[src_v.at[j + NBUF]],
                                     rows.at[b], sems[b])

                for kk in range(CH // 16):
                    idx = dst_v[j, pl.ds(kk * 16, 16)]
                    plsc.addupdate_scatter(hist, [idx], ones16)
            return carry

        lax.fori_loop(0, NCHUNK // NBUF, group, 0)

        pltpu.sync_copy(hist, cnt_hbm.at[c, s])
        plsc.subcore_barrier()
        pltpu.sync_copy(acc_sh.at[pl.ds(s * ROWS_PT, ROWS_PT)],
                        part_hbm.at[c, s])

    return k(x, srcr, dstr, zrows, zcnt)


def _tc_finish(parts, cnts, x, wlt, wrt, b):
    B = 1000

    def body(p_ref, c_ref, x_ref, wlt_ref, wrt_ref, b_ref, o_ref):
        p = p_ref[...].astype(jnp.float32)
        summed = p[0] + p[1]
        cnt = jnp.sum(c_ref[...], axis=1, keepdims=True)
        mean = summed / jnp.maximum(cnt, 1.0)
        o_ref[...] = (
            jnp.dot(mean, wlt_ref[...], preferred_element_type=jnp.float32)
            + jnp.dot(x_ref[...], wrt_ref[...],
                      preferred_element_type=jnp.float32)
            + b_ref[...]
        )

    return pl.pallas_call(
        body,
        grid=(N // B,),
        in_specs=[
            pl.BlockSpec((NC, B, D), lambda i: (0, i, 0)),
            pl.BlockSpec((B, NW), lambda i: (i, 0)),
            pl.BlockSpec((B, D), lambda i: (i, 0)),
            pl.BlockSpec((D, D), lambda i: (0, 0)),
            pl.BlockSpec((D, D), lambda i: (0, 0)),
            pl.BlockSpec((1, D), lambda i: (0, 0)),
        ],
        out_specs=pl.BlockSpec((B, D), lambda i: (i, 0)),
        out_shape=jax.ShapeDtypeStruct((N, D), jnp.float32),
    )(parts, cnts, x, wlt, wrt, b)


def kernel(x, edge_index, W_l, b_l, W_r, training):
    src = edge_index[0].astype(jnp.int32).reshape(NW, NCHUNK, CH)
    dst = edge_index[1].astype(jnp.int32).reshape(NW, NCHUNK, CH)
    zrows = jnp.zeros((ROWS_PT, D), jnp.bfloat16)
    zcnt = jnp.zeros((N,), jnp.float32)
    xb = x.astype(jnp.bfloat16)
    parts, cnts = _sc_segment_sum(xb, src, dst, zrows, zcnt)
    parts = parts.reshape(NC, N, D)
    cnts = cnts.reshape(NW, N).T
    return _tc_finish(parts, cnts, x, W_l.T, W_r.T, b_l.reshape(1, D))
